# baseline (speedup 1.0000x reference)
import jax
import jax.numpy as jnp
from jax.experimental import pallas as pl
from jax.experimental.pallas import tpu as pltpu

N_CLASSES = 32
CCE_W = 1.0
DICE_W = 0.5
EPS = 1e-08
BR = 4096


def _loss_kernel(pred_ref, gt_ref, w_ref, out_ref,
                 hist_gt_ref, hist_pred_ref, hist_tp_ref, cce_ref):
    i = pl.program_id(0)
    nsteps = pl.num_programs(0)

    @pl.when(i == 0)
    def _init():
        hist_gt_ref[...] = jnp.zeros_like(hist_gt_ref)
        hist_pred_ref[...] = jnp.zeros_like(hist_pred_ref)
        hist_tp_ref[...] = jnp.zeros_like(hist_tp_ref)
        cce_ref[0, 0] = 0.0

    x = pred_ref[...].T.reshape(4, N_CLASSES, BR)
    g = gt_ref[...].T.reshape(4, N_CLASSES, BR)
    w = w_ref[...].reshape(1, N_CLASSES, 1)

    e = jnp.exp(x)
    s = jnp.sum(e, axis=1, keepdims=True)
    q = e / s

    lse = jnp.log(jnp.sum(jnp.exp(q), axis=1, keepdims=True))
    gw = g * w
    sgw = jnp.sum(gw, axis=1, keepdims=True)
    cce_ref[0, 0] += jnp.sum(lse * sgw) - jnp.sum(gw * q)

    m = jnp.max(x, axis=1, keepdims=True)
    gm = jnp.max(g, axis=1, keepdims=True)
    pred_oh = (x == m).astype(jnp.float32)
    gt_oh = (g == gm).astype(jnp.float32)
    tp_oh = gt_oh * pred_oh
    hist_gt_ref[...] += jnp.sum(jnp.sum(gt_oh, axis=0), axis=1, keepdims=True)
    hist_pred_ref[...] += jnp.sum(jnp.sum(pred_oh, axis=0), axis=1, keepdims=True)
    hist_tp_ref[...] += jnp.sum(jnp.sum(tp_oh, axis=0), axis=1, keepdims=True)

    @pl.when(i == nsteps - 1)
    def _finish():
        tp = hist_tp_ref[...]
        denom = hist_gt_ref[...] + hist_pred_ref[...] - tp
        dice = (tp + EPS) / (denom + EPS)
        dice_loss = jnp.sum((1.0 - dice) * w_ref[...]) / N_CLASSES
        n_total = nsteps * BR * 4
        cce_loss = cce_ref[0, 0] / n_total
        total = cce_loss * CCE_W + dice_loss * DICE_W
        out_ref[...] = jnp.full((1, 1), total, dtype=jnp.float32)


def kernel(predictions, ground_truth, class_weights):
    n, c = predictions.shape
    xp = predictions.reshape(n // 4, 4 * c)
    gp = ground_truth.reshape(n // 4, 4 * c)
    w2 = class_weights.reshape(c, 1)
    grid = (n // 4 // BR,)
    out = pl.pallas_call(
        _loss_kernel,
        grid=grid,
        in_specs=[
            pl.BlockSpec((BR, 4 * c), lambda i: (i, 0)),
            pl.BlockSpec((BR, 4 * c), lambda i: (i, 0)),
            pl.BlockSpec((c, 1), lambda i: (0, 0)),
        ],
        out_specs=pl.BlockSpec((1, 1), lambda i: (0, 0)),
        out_shape=jax.ShapeDtypeStruct((1, 1), jnp.float32),
        scratch_shapes=[
            pltpu.VMEM((c, 1), jnp.float32),
            pltpu.VMEM((c, 1), jnp.float32),
            pltpu.VMEM((c, 1), jnp.float32),
            pltpu.SMEM((1, 1), jnp.float32),
        ],
    )(xp, gp, w2)
    return out.reshape(())


# speedup vs baseline: 1.1964x; 1.1964x over previous
import jax
import jax.numpy as jnp
from jax import lax
from jax.experimental import pallas as pl
from jax.experimental.pallas import tpu as pltpu

N_CLASSES = 32
CCE_W = 1.0
DICE_W = 0.5
EPS = 1e-08
BN = 16384


def _loss_kernel(pred_ref, gt_ref, w_ref, out_ref, conf_ref, cce_ref):
    i = pl.program_id(0)
    nsteps = pl.num_programs(0)

    @pl.when(i == 0)
    def _init():
        conf_ref[...] = jnp.zeros_like(conf_ref)
        cce_ref[0, 0] = 0.0

    x = pred_ref[...].T
    g = gt_ref[...].T
    wcol = w_ref[...]
    ones8 = jnp.ones((8, N_CLASSES), jnp.float32)

    e = jnp.exp(x)
    s = lax.dot_general(ones8, e, (((1,), (0,)), ((), ())),
                        preferred_element_type=jnp.float32)[0:1]
    q = e / s

    eq = jnp.exp(q)
    t = lax.dot_general(ones8, eq, (((1,), (0,)), ((), ())),
                        preferred_element_type=jnp.float32)[0:1]
    lse = jnp.log(t)
    gw = g * wcol
    sgw = lax.dot_general(ones8, gw, (((1,), (0,)), ((), ())),
                          preferred_element_type=jnp.float32)[0:1]
    cce_ref[0, 0] += jnp.sum(lse * sgw) - jnp.sum(gw * q)

    m = jnp.max(x, axis=0, keepdims=True)
    gm = jnp.max(g, axis=0, keepdims=True)
    pred_oh = (x == m).astype(jnp.float32)
    gt_oh = (g == gm).astype(jnp.float32)
    conf_ref[...] += lax.dot_general(gt_oh, pred_oh, (((1,), (1,)), ((), ())),
                                     preferred_element_type=jnp.float32)

    @pl.when(i == nsteps - 1)
    def _finish():
        conf = conf_ref[...]
        eye = (jax.lax.broadcasted_iota(jnp.int32, conf.shape, 0)
               == jax.lax.broadcasted_iota(jnp.int32, conf.shape, 1))
        tp = jnp.sum(jnp.where(eye, conf, 0.0), axis=0, keepdims=True)
        rows = jnp.sum(conf.T, axis=0, keepdims=True)
        cols = jnp.sum(conf, axis=0, keepdims=True)
        denom = rows + cols - tp
        dice = (tp + EPS) / (denom + EPS)
        dice_loss = jnp.sum((1.0 - dice) * wcol.T) / N_CLASSES
        n_total = nsteps * BN
        cce_loss = cce_ref[0, 0] / n_total
        total = cce_loss * CCE_W + dice_loss * DICE_W
        out_ref[...] = jnp.full((1, 1), total, dtype=jnp.float32)


def kernel(predictions, ground_truth, class_weights):
    n, c = predictions.shape
    w2 = class_weights.reshape(c, 1)
    grid = (n // BN,)
    out = pl.pallas_call(
        _loss_kernel,
        grid=grid,
        in_specs=[
            pl.BlockSpec((BN, c), lambda i: (i, 0)),
            pl.BlockSpec((BN, c), lambda i: (i, 0)),
            pl.BlockSpec((c, 1), lambda i: (0, 0)),
        ],
        out_specs=pl.BlockSpec((1, 1), lambda i: (0, 0)),
        out_shape=jax.ShapeDtypeStruct((1, 1), jnp.float32),
        scratch_shapes=[
            pltpu.VMEM((c, c), jnp.float32),
            pltpu.SMEM((1, 1), jnp.float32),
        ],
    )(predictions, ground_truth, w2)
    return out.reshape(())


# speedup vs baseline: 7.6838x; 6.4224x over previous
import jax
import jax.numpy as jnp
from jax import lax
from jax.experimental import pallas as pl
from jax.experimental.pallas import tpu as pltpu

N_CLASSES = 32
CCE_W = 1.0
DICE_W = 0.5
EPS = 1e-08
BN = 16384


def _loss_kernel(pred_ref, gt_ref, w_ref, out_ref, conf_ref, cce_ref):
    i = pl.program_id(0)
    nsteps = pl.num_programs(0)

    @pl.when(i == 0)
    def _init():
        conf_ref[...] = jnp.zeros_like(conf_ref)
        cce_ref[0, 0] = 0.0

    x = pred_ref[...]
    g = gt_ref[...]
    wcol = w_ref[...]
    ones8 = jnp.ones((8, N_CLASSES), jnp.float32)

    e = jnp.exp(x)
    s = lax.dot_general(ones8, e, (((1,), (0,)), ((), ())),
                        preferred_element_type=jnp.float32)[0:1]
    q = e / s

    eq = jnp.exp(q)
    t = lax.dot_general(ones8, eq, (((1,), (0,)), ((), ())),
                        preferred_element_type=jnp.float32)[0:1]
    lse = jnp.log(t)
    gw = g * wcol
    sgw = lax.dot_general(ones8, gw, (((1,), (0,)), ((), ())),
                          preferred_element_type=jnp.float32)[0:1]
    cce_ref[0, 0] += jnp.sum(lse * sgw) - jnp.sum(gw * q)

    m = jnp.max(x, axis=0, keepdims=True)
    gm = jnp.max(g, axis=0, keepdims=True)
    pred_oh = (x == m).astype(jnp.float32)
    gt_oh = (g == gm).astype(jnp.float32)
    conf_ref[...] += lax.dot_general(gt_oh, pred_oh, (((1,), (1,)), ((), ())),
                                     preferred_element_type=jnp.float32)

    @pl.when(i == nsteps - 1)
    def _finish():
        conf = conf_ref[...]
        eye = (jax.lax.broadcasted_iota(jnp.int32, conf.shape, 0)
               == jax.lax.broadcasted_iota(jnp.int32, conf.shape, 1))
        tp = jnp.sum(jnp.where(eye, conf, 0.0), axis=0, keepdims=True)
        rows = jnp.sum(conf.T, axis=0, keepdims=True)
        cols = jnp.sum(conf, axis=0, keepdims=True)
        denom = rows + cols - tp
        dice = (tp + EPS) / (denom + EPS)
        dice_loss = jnp.sum((1.0 - dice) * wcol.T) / N_CLASSES
        n_total = nsteps * BN
        cce_loss = cce_ref[0, 0] / n_total
        total = cce_loss * CCE_W + dice_loss * DICE_W
        out_ref[...] = jnp.full((1, 1), total, dtype=jnp.float32)


def kernel(predictions, ground_truth, class_weights):
    n, c = predictions.shape
    xT = predictions.T
    gT = ground_truth.T
    w2 = class_weights.reshape(c, 1)
    grid = (n // BN,)
    out = pl.pallas_call(
        _loss_kernel,
        grid=grid,
        in_specs=[
            pl.BlockSpec((c, BN), lambda i: (0, i)),
            pl.BlockSpec((c, BN), lambda i: (0, i)),
            pl.BlockSpec((c, 1), lambda i: (0, 0)),
        ],
        out_specs=pl.BlockSpec((1, 1), lambda i: (0, 0)),
        out_shape=jax.ShapeDtypeStruct((1, 1), jnp.float32),
        scratch_shapes=[
            pltpu.VMEM((c, c), jnp.float32),
            pltpu.SMEM((1, 1), jnp.float32),
        ],
    )(xT, gT, w2)
    return out.reshape(())


# speedup vs baseline: 7.8232x; 1.0181x over previous
import jax
import jax.numpy as jnp
from jax import lax
from jax.experimental import pallas as pl
from jax.experimental.pallas import tpu as pltpu

N_CLASSES = 32
CCE_W = 1.0
DICE_W = 0.5
EPS = 1e-08
BN = 32768


def _loss_kernel(pred_ref, gt_ref, w_ref, out_ref, conf_ref, cce_ref):
    i = pl.program_id(0)
    nsteps = pl.num_programs(0)

    @pl.when(i == 0)
    def _init():
        conf_ref[...] = jnp.zeros_like(conf_ref)
        cce_ref[0, 0] = 0.0

    x = pred_ref[...]
    g = gt_ref[...]
    wcol = w_ref[...]
    ones8 = jnp.ones((8, N_CLASSES), jnp.float32)

    e = jnp.exp(x)
    s = lax.dot_general(ones8, e, (((1,), (0,)), ((), ())),
                        preferred_element_type=jnp.float32)[0:1]
    q = e / s

    eq = jnp.exp(q)
    t = lax.dot_general(ones8, eq, (((1,), (0,)), ((), ())),
                        preferred_element_type=jnp.float32)[0:1]
    lse = jnp.log(t)
    gw = g * wcol
    sgw = lax.dot_general(ones8, gw, (((1,), (0,)), ((), ())),
                          preferred_element_type=jnp.float32)[0:1]
    cce_ref[0, 0] += jnp.sum(lse * sgw) - jnp.sum(gw * q)

    m = jnp.max(x, axis=0, keepdims=True)
    gm = jnp.max(g, axis=0, keepdims=True)
    pred_oh = (x == m).astype(jnp.float32)
    gt_oh = (g == gm).astype(jnp.float32)
    conf_ref[...] += lax.dot_general(gt_oh, pred_oh, (((1,), (1,)), ((), ())),
                                     preferred_element_type=jnp.float32)

    @pl.when(i == nsteps - 1)
    def _finish():
        conf = conf_ref[...]
        eye = (jax.lax.broadcasted_iota(jnp.int32, conf.shape, 0)
               == jax.lax.broadcasted_iota(jnp.int32, conf.shape, 1))
        tp = jnp.sum(jnp.where(eye, conf, 0.0), axis=0, keepdims=True)
        rows = jnp.sum(conf.T, axis=0, keepdims=True)
        cols = jnp.sum(conf, axis=0, keepdims=True)
        denom = rows + cols - tp
        dice = (tp + EPS) / (denom + EPS)
        dice_loss = jnp.sum((1.0 - dice) * wcol.T) / N_CLASSES
        n_total = nsteps * BN
        cce_loss = cce_ref[0, 0] / n_total
        total = cce_loss * CCE_W + dice_loss * DICE_W
        out_ref[...] = jnp.full((1, 1), total, dtype=jnp.float32)


def kernel(predictions, ground_truth, class_weights):
    n, c = predictions.shape
    xT = predictions.T
    gT = ground_truth.T
    w2 = class_weights.reshape(c, 1)
    grid = (n // BN,)
    out = pl.pallas_call(
        _loss_kernel,
        grid=grid,
        in_specs=[
            pl.BlockSpec((c, BN), lambda i: (0, i)),
            pl.BlockSpec((c, BN), lambda i: (0, i)),
            pl.BlockSpec((c, 1), lambda i: (0, 0)),
        ],
        out_specs=pl.BlockSpec((1, 1), lambda i: (0, 0)),
        out_shape=jax.ShapeDtypeStruct((1, 1), jnp.float32),
        scratch_shapes=[
            pltpu.VMEM((c, c), jnp.float32),
            pltpu.SMEM((1, 1), jnp.float32),
        ],
    )(xT, gT, w2)
    return out.reshape(())
